# Initial kernel scaffold; baseline (speedup 1.0000x reference)
import functools

import jax
import jax.numpy as jnp
from jax import lax
from jax.experimental import pallas as pl
from jax.experimental.pallas import tpu as pltpu
from jax.experimental.pallas import tpu_sc as plsc

N = 10000
E = 320000
D = 128
Z = 0.8
EPS = 1e-5

NC = 2
NS = 16
NW = NC * NS
EPAD = 327680
EPW = EPAD // NW
CHUNK = 128
NCHUNK = EPW // CHUNK
RPT = N // NS

_BLK = 1000
_GRID = N // _BLK


@functools.partial(
    pl.kernel,
    out_type=[
        jax.ShapeDtypeStruct((NC, N, D), jnp.float32),
        jax.ShapeDtypeStruct((NW, N), jnp.float32),
    ],
    mesh=plsc.VectorSubcoreMesh(core_axis_name="c", subcore_axis_name="s"),
    scratch_types=[
        pltpu.VMEM((EPW,), jnp.int32),
        pltpu.VMEM((EPW,), jnp.int32),
        pltpu.VMEM((EPW,), jnp.float32),
        pltpu.VMEM((CHUNK,), jnp.int32),
        pltpu.VMEM((CHUNK, D), jnp.float32),
        pltpu.VMEM((N,), jnp.float32),
        pltpu.VMEM_SHARED((N, D), jnp.float32),
        pltpu.SemaphoreType.DMA,
    ],
)
def _sc_spmm(row_hbm, col_hbm, ew_hbm, x_hbm, outp_hbm, degp_hbm,
             row_v, col_v, ew_v, sidx_v, rows_v, deg_v, part_sh, sem):
    cid = lax.axis_index("c")
    sid = lax.axis_index("s")
    wid = cid * NS + sid
    base = wid * EPW

    zvec = jnp.zeros((16,), jnp.float32)

    @pl.loop(0, N, step=16)
    def _(i):
        deg_v[pl.ds(i, 16)] = zvec

    @pl.loop(0, CHUNK)
    def _(i):
        for j in range(D // 16):
            rows_v[i, pl.ds(j * 16, 16)] = zvec

    for t in range(4):
        pltpu.sync_copy(rows_v, part_sh.at[pl.ds(sid * RPT + t * CHUNK, CHUNK)])
    pltpu.sync_copy(rows_v.at[pl.ds(0, RPT - 4 * CHUNK)],
                    part_sh.at[pl.ds(sid * RPT + 4 * CHUNK, RPT - 4 * CHUNK)])

    pltpu.sync_copy(row_hbm.at[pl.ds(base, EPW)], row_v)
    pltpu.sync_copy(col_hbm.at[pl.ds(base, EPW)], col_v)
    pltpu.sync_copy(ew_hbm.at[pl.ds(base, EPW)], ew_v)

    plsc.subcore_barrier()

    @pl.loop(0, NCHUNK)
    def _(c):
        off = c * CHUNK
        pltpu.async_copy(x_hbm.at[col_v.at[pl.ds(off, CHUNK)]], rows_v, sem).wait()

        for g in range(CHUNK // 16):
            rvec = row_v[pl.ds(off + g * 16, 16)]
            sidx_v[pl.ds(g * 16, 16)] = rvec
            plsc.addupdate_scatter(deg_v, [rvec], ew_v[pl.ds(off + g * 16, 16)])

        @pl.loop(0, CHUNK)
        def _(e):
            splat = plsc.load_gather(ew_v, [jnp.full((16,), off + e, jnp.int32)])
            for j in range(D // 16):
                rows_v[e, pl.ds(j * 16, 16)] = rows_v[e, pl.ds(j * 16, 16)] * splat

        pltpu.sync_copy(rows_v, part_sh.at[sidx_v], add=True)

    plsc.subcore_barrier()

    pltpu.sync_copy(part_sh.at[pl.ds(sid * RPT, RPT)],
                    outp_hbm.at[cid, pl.ds(sid * RPT, RPT)])
    pltpu.sync_copy(deg_v, degp_hbm.at[wid])


def _mix(mf, a1, a0):
    sel = a0 + mf * (a1 - a0)
    return (1.0 - Z) * (a0 + a1) + (2.0 * Z - 1.0) * sel


def _tc_mix_body(x_ref, m_ref, w0_ref, b0_ref, w1_ref, b1_ref, o_ref):
    x = x_ref[...]
    dn = (((1,), (1,)), ((), ()))
    h1 = jax.nn.relu(lax.dot_general(x, w1_ref[...], dn,
                                     preferred_element_type=jnp.float32) + b1_ref[...])
    h0 = jax.nn.relu(lax.dot_general(x, w0_ref[...], dn,
                                     preferred_element_type=jnp.float32) + b0_ref[...])
    o_ref[...] = _mix(m_ref[...], h1, h0)


def _tc_mix(x_, mf, Wt0, bt0, Wt1, bt1):
    return pl.pallas_call(
        _tc_mix_body,
        grid=(_GRID,),
        in_specs=[
            pl.BlockSpec((_BLK, D), lambda i: (i, 0)),
            pl.BlockSpec((_BLK, 1), lambda i: (i, 0)),
            pl.BlockSpec((D, D), lambda i: (0, 0)),
            pl.BlockSpec((1, D), lambda i: (0, 0)),
            pl.BlockSpec((D, D), lambda i: (0, 0)),
            pl.BlockSpec((1, D), lambda i: (0, 0)),
        ],
        out_specs=pl.BlockSpec((_BLK, D), lambda i: (i, 0)),
        out_shape=jax.ShapeDtypeStruct((N, D), jnp.float32),
    )(x_, mf, Wt0, bt0, Wt1, bt1)


def _tc_stats_body(p_ref, d_ref, agg_ref, st_ref):
    i = pl.program_id(0)
    deg = jnp.sum(d_ref[...], axis=0)
    deg = jnp.where(deg < 0.5, deg + 1.0, deg)
    agg = (p_ref[0] + p_ref[1]) * (1.0 / deg)[:, None]
    agg_ref[...] = agg
    st = jnp.concatenate([jnp.sum(agg, axis=0, keepdims=True),
                          jnp.sum(agg * agg, axis=0, keepdims=True)], axis=0)

    @pl.when(i == 0)
    def _():
        st_ref[...] = st

    @pl.when(i > 0)
    def _():
        st_ref[...] = st_ref[...] + st


def _tc_stats(outp, degp):
    return pl.pallas_call(
        _tc_stats_body,
        grid=(_GRID,),
        in_specs=[
            pl.BlockSpec((NC, _BLK, D), lambda i: (0, i, 0)),
            pl.BlockSpec((NW, _BLK), lambda i: (0, i)),
        ],
        out_specs=[
            pl.BlockSpec((_BLK, D), lambda i: (i, 0)),
            pl.BlockSpec((2, D), lambda i: (0, 0)),
        ],
        out_shape=[
            jax.ShapeDtypeStruct((N, D), jnp.float32),
            jax.ShapeDtypeStruct((2, D), jnp.float32),
        ],
    )(outp, degp)


def _tc_final_body(agg_ref, st_ref, x_ref, m_ref,
                   a1_ref, b1m_ref, bc1_ref, a0_ref, b0m_ref, bc0_ref,
                   gw_ref, gb_ref, ms_ref, o_ref):
    ms = ms_ref[...]
    s1 = st_ref[pl.ds(0, 1), :]
    s2 = st_ref[pl.ds(1, 1), :]
    mean = s1 * (1.0 / N)
    var = s2 * (1.0 / N) - (2.0 * ms - ms * ms) * mean * mean
    agg = agg_ref[...]
    xn = gw_ref[...] * (agg - ms * mean) * lax.rsqrt(var + EPS) + gb_ref[...]
    xv = x_ref[...]
    dn = (((1,), (1,)), ((), ()))
    y1 = (lax.dot_general(xn, a1_ref[...], dn, preferred_element_type=jnp.float32)
          + lax.dot_general(xv, b1m_ref[...], dn, preferred_element_type=jnp.float32)
          + bc1_ref[...])
    y0 = (lax.dot_general(xn, a0_ref[...], dn, preferred_element_type=jnp.float32)
          + lax.dot_general(xv, b0m_ref[...], dn, preferred_element_type=jnp.float32)
          + bc0_ref[...])
    o_ref[...] = _mix(m_ref[...], y1, y0)


def _tc_final(agg, st, x_, mf, A1, B1, bc1, A0, B0, bc0, gw, gb, ms):
    wspec = pl.BlockSpec((D, D), lambda i: (0, 0))
    vspec = pl.BlockSpec((1, D), lambda i: (0, 0))
    return pl.pallas_call(
        _tc_final_body,
        grid=(_GRID,),
        in_specs=[
            pl.BlockSpec((_BLK, D), lambda i: (i, 0)),
            pl.BlockSpec((2, D), lambda i: (0, 0)),
            pl.BlockSpec((_BLK, D), lambda i: (i, 0)),
            pl.BlockSpec((_BLK, 1), lambda i: (i, 0)),
            wspec, wspec, vspec, wspec, wspec, vspec,
            vspec, vspec, vspec,
        ],
        out_specs=pl.BlockSpec((_BLK, D), lambda i: (i, 0)),
        out_shape=jax.ShapeDtypeStruct((N, D), jnp.float32),
    )(agg, st, x_, mf, A1, B1, bc1, A0, B0, bc0, gw, gb, ms)


def kernel(x_, edge_index, edge_weight, mask,
           Wt0, bt0, Wt1, bt1, Wc0, bc0, Wc1, bc1,
           gn_weight, gn_bias, gn_mean_scale):
    row = edge_index[0]
    col = edge_index[1]
    pad = EPAD - E
    zi = jnp.zeros((pad,), jnp.int32)
    rowp = jnp.concatenate([row, zi])
    colp = jnp.concatenate([col, zi])
    ewp = jnp.concatenate([edge_weight, jnp.zeros((pad,), jnp.float32)])
    mf = mask.astype(jnp.float32)

    x_mix = _tc_mix(x_, mf, Wt0, bt0.reshape(1, D), Wt1, bt1.reshape(1, D))
    outp, degp = _sc_spmm(rowp, colp, ewp, x_mix)
    agg, st = _tc_stats(outp, degp)
    return _tc_final(
        agg, st, x_, mf,
        Wc1[:, :D], Wc1[:, D:], bc1.reshape(1, D),
        Wc0[:, :D], Wc0[:, D:], bc0.reshape(1, D),
        gn_weight.reshape(1, D), gn_bias.reshape(1, D),
        gn_mean_scale.reshape(1, D))


# speedup vs baseline: 4.7433x; 4.7433x over previous
import dataclasses
import functools

import jax
import jax.numpy as jnp
from jax import lax
from jax.experimental import pallas as pl
from jax.experimental.pallas import tpu as pltpu
from jax.experimental.pallas import tpu_sc as plsc

N = 10000
E = 320000
D = 128
Z = 0.8
EPS = 1e-5

NC = 2
NS = 16
NW = NC * NS
EPAD = 327680
EPW = EPAD // NW
CHUNK = 128
NCHUNK = EPW // CHUNK
NP = 10240
RPT = NP // NS

_BLK = 1000
_GRID = N // _BLK


_SC_CP = pltpu.CompilerParams()
if "needs_layout_passes" in pltpu.CompilerParams.__dataclass_fields__:
    _SC_CP = dataclasses.replace(_SC_CP, needs_layout_passes=False)


@functools.lru_cache(maxsize=1)
def _sc_spmm_build():
    return functools.partial(
        pl.kernel,
        compiler_params=_SC_CP,
        out_type=[
            jax.ShapeDtypeStruct((NC, NP, D), jnp.float32),
            jax.ShapeDtypeStruct((NW * N,), jnp.float32),
        ],
        mesh=plsc.VectorSubcoreMesh(core_axis_name="c", subcore_axis_name="s"),
        scratch_types=[
            pltpu.VMEM((2, CHUNK), jnp.int32),
            pltpu.VMEM((CHUNK,), jnp.float32),
            pltpu.VMEM((CHUNK,), jnp.int32),
            pltpu.VMEM((CHUNK, D), jnp.float32),
            pltpu.VMEM((N,), jnp.float32),
            pltpu.VMEM_SHARED((NP, D), jnp.float32),
            pltpu.SemaphoreType.DMA,
        ],
    )(_sc_spmm_body)


def _sc_spmm_body(ei_hbm, ew_hbm, x_hbm, outp_hbm, degp_hbm,
                  eidx_v, ew_v, sidx_v, rows_v, deg_v, part_sh, sem):
    cid = lax.axis_index("c")
    sid = lax.axis_index("s")
    wid = cid * NS + sid
    base = wid * EPW

    zvec = jnp.zeros((16,), jnp.float32)

    @pl.loop(0, N, step=16)
    def _(i):
        deg_v[pl.ds(i, 16)] = zvec

    @pl.loop(0, CHUNK)
    def _(i):
        for j in range(D // 16):
            rows_v[i, pl.ds(j * 16, 16)] = zvec

    for t in range(RPT // CHUNK):
        pltpu.sync_copy(rows_v, part_sh.at[pl.ds(sid * RPT + t * CHUNK, CHUNK)])

    plsc.subcore_barrier()

    @pl.loop(0, NCHUNK)
    def _(c):
        off = base + c * CHUNK
        pltpu.sync_copy(ei_hbm.at[:, pl.ds(off, CHUNK)], eidx_v)
        pltpu.sync_copy(ew_hbm.at[pl.ds(off, CHUNK)], ew_v)
        pltpu.async_copy(x_hbm.at[eidx_v.at[1]], rows_v, sem).wait()

        for g in range(CHUNK // 16):
            rvec = eidx_v[0, pl.ds(g * 16, 16)]
            sidx_v[pl.ds(g * 16, 16)] = rvec
            plsc.addupdate_scatter(deg_v, [rvec], ew_v[pl.ds(g * 16, 16)])

        @pl.loop(0, CHUNK)
        def _(e):
            splat = plsc.load_gather(ew_v, [jnp.full((16,), e, jnp.int32)])
            for j in range(D // 16):
                rows_v[e, pl.ds(j * 16, 16)] = rows_v[e, pl.ds(j * 16, 16)] * splat

        pltpu.sync_copy(rows_v, part_sh.at[sidx_v], add=True)

    plsc.subcore_barrier()

    pltpu.sync_copy(part_sh.at[pl.ds(sid * RPT, RPT)],
                    outp_hbm.at[cid, pl.ds(sid * RPT, RPT)])
    pltpu.sync_copy(deg_v, degp_hbm.at[pl.ds(wid * N, N)])


def _mix(mf, a1, a0):
    sel = a0 + mf * (a1 - a0)
    return (1.0 - Z) * (a0 + a1) + (2.0 * Z - 1.0) * sel


def _tc_mix_body(x_ref, m_ref, w0_ref, b0_ref, w1_ref, b1_ref, o_ref):
    x = x_ref[...]
    dn = (((1,), (1,)), ((), ()))
    h1 = jax.nn.relu(lax.dot_general(x, w1_ref[...], dn,
                                     preferred_element_type=jnp.float32) + b1_ref[...])
    h0 = jax.nn.relu(lax.dot_general(x, w0_ref[...], dn,
                                     preferred_element_type=jnp.float32) + b0_ref[...])
    o_ref[...] = _mix(m_ref[...], h1, h0)


def _tc_mix(x_, mf, Wt0, bt0, Wt1, bt1):
    return pl.pallas_call(
        _tc_mix_body,
        grid=(_GRID,),
        in_specs=[
            pl.BlockSpec((_BLK, D), lambda i: (i, 0)),
            pl.BlockSpec((_BLK, 1), lambda i: (i, 0)),
            pl.BlockSpec((D, D), lambda i: (0, 0)),
            pl.BlockSpec((1, D), lambda i: (0, 0)),
            pl.BlockSpec((D, D), lambda i: (0, 0)),
            pl.BlockSpec((1, D), lambda i: (0, 0)),
        ],
        out_specs=pl.BlockSpec((_BLK, D), lambda i: (i, 0)),
        out_shape=jax.ShapeDtypeStruct((N, D), jnp.float32),
    )(x_, mf, Wt0, bt0, Wt1, bt1)


def _tc_stats_body(p_ref, d_ref, agg_ref, st_ref):
    i = pl.program_id(0)
    deg = jnp.sum(d_ref[...], axis=1)
    deg = jnp.where(deg < 0.5, deg + 1.0, deg)
    agg = (p_ref[0] + p_ref[1]) * (1.0 / deg)[:, None]
    agg_ref[...] = agg
    st = jnp.concatenate([jnp.sum(agg, axis=0, keepdims=True),
                          jnp.sum(agg * agg, axis=0, keepdims=True)], axis=0)

    @pl.when(i == 0)
    def _():
        st_ref[...] = st

    @pl.when(i > 0)
    def _():
        st_ref[...] = st_ref[...] + st


def _tc_stats(outp, degp):
    return pl.pallas_call(
        _tc_stats_body,
        grid=(_GRID,),
        in_specs=[
            pl.BlockSpec((NC, _BLK, D), lambda i: (0, i, 0)),
            pl.BlockSpec((_BLK, NW), lambda i: (i, 0)),
        ],
        out_specs=[
            pl.BlockSpec((_BLK, D), lambda i: (i, 0)),
            pl.BlockSpec((2, D), lambda i: (0, 0)),
        ],
        out_shape=[
            jax.ShapeDtypeStruct((N, D), jnp.float32),
            jax.ShapeDtypeStruct((2, D), jnp.float32),
        ],
    )(outp, degp)


def _tc_final_body(agg_ref, st_ref, x_ref, m_ref,
                   a1_ref, b1m_ref, bc1_ref, a0_ref, b0m_ref, bc0_ref,
                   gw_ref, gb_ref, ms_ref, o_ref):
    ms = ms_ref[...]
    s1 = st_ref[pl.ds(0, 1), :]
    s2 = st_ref[pl.ds(1, 1), :]
    mean = s1 * (1.0 / N)
    var = s2 * (1.0 / N) - (2.0 * ms - ms * ms) * mean * mean
    agg = agg_ref[...]
    xn = gw_ref[...] * (agg - ms * mean) * lax.rsqrt(var + EPS) + gb_ref[...]
    xv = x_ref[...]
    dn = (((1,), (1,)), ((), ()))
    y1 = (lax.dot_general(xn, a1_ref[...], dn, preferred_element_type=jnp.float32)
          + lax.dot_general(xv, b1m_ref[...], dn, preferred_element_type=jnp.float32)
          + bc1_ref[...])
    y0 = (lax.dot_general(xn, a0_ref[...], dn, preferred_element_type=jnp.float32)
          + lax.dot_general(xv, b0m_ref[...], dn, preferred_element_type=jnp.float32)
          + bc0_ref[...])
    o_ref[...] = _mix(m_ref[...], y1, y0)


def _tc_final(agg, st, x_, mf, A1, B1, bc1, A0, B0, bc0, gw, gb, ms):
    wspec = pl.BlockSpec((D, D), lambda i: (0, 0))
    vspec = pl.BlockSpec((1, D), lambda i: (0, 0))
    return pl.pallas_call(
        _tc_final_body,
        grid=(_GRID,),
        in_specs=[
            pl.BlockSpec((_BLK, D), lambda i: (i, 0)),
            pl.BlockSpec((2, D), lambda i: (0, 0)),
            pl.BlockSpec((_BLK, D), lambda i: (i, 0)),
            pl.BlockSpec((_BLK, 1), lambda i: (i, 0)),
            wspec, wspec, vspec, wspec, wspec, vspec,
            vspec, vspec, vspec,
        ],
        out_specs=pl.BlockSpec((_BLK, D), lambda i: (i, 0)),
        out_shape=jax.ShapeDtypeStruct((N, D), jnp.float32),
    )(agg, st, x_, mf, A1, B1, bc1, A0, B0, bc0, gw, gb, ms)


def kernel(x_, edge_index, edge_weight, mask,
           Wt0, bt0, Wt1, bt1, Wc0, bc0, Wc1, bc1,
           gn_weight, gn_bias, gn_mean_scale):
    pad = EPAD - E
    eip = jnp.concatenate([edge_index, jnp.zeros((2, pad), jnp.int32)], axis=1)
    ewp = jnp.concatenate([edge_weight, jnp.zeros((pad,), jnp.float32)])
    mf = mask.astype(jnp.float32)

    x_mix = _tc_mix(x_, mf, Wt0, bt0.reshape(1, D), Wt1, bt1.reshape(1, D))
    outp, degp = _sc_spmm_build()(eip, ewp, x_mix)
    agg, st = _tc_stats(outp, degp.reshape(NW, N).T)
    return _tc_final(
        agg, st, x_, mf,
        Wc1[:, :D], Wc1[:, D:], bc1.reshape(1, D),
        Wc0[:, :D], Wc0[:, D:], bc0.reshape(1, D),
        gn_weight.reshape(1, D), gn_bias.reshape(1, D),
        gn_mean_scale.reshape(1, D))
